# initial kernel scaffold (unmeasured)
import jax
import jax.numpy as jnp
from jax import lax
from jax.experimental import pallas as pl
from jax.experimental.pallas import tpu as pltpu

T = 1024
D = 2048
V_LOCAL = 16384
V_TILE = 2048
N_TILES = V_LOCAL // V_TILE


def kernel(x, W, labels):
    labels2d = labels.reshape(T, 1)

    def body(x_ref, w_ref, lab_ref, out_ref,
             xb_ref, s_ref, t_ref, comm_ref, recv_ref,
             send_sem, recv_sem):
        step = pl.program_id(0)
        my_x = lax.axis_index("x")
        my_y = lax.axis_index("y")

        @pl.when(step == 0)
        def _():
            xb_ref[...] = x_ref[...].astype(jnp.bfloat16)
            s_ref[...] = jnp.zeros((T, 1), jnp.float32)
            t_ref[...] = jnp.zeros((T, 1), jnp.float32)

        logits = jnp.dot(
            xb_ref[...],
            w_ref[...].astype(jnp.bfloat16),
            preferred_element_type=jnp.float32,
        )

        s_ref[...] += jnp.sum(jnp.exp(logits), axis=1, keepdims=True)

        base = my_y * V_LOCAL + step * V_TILE
        loc = lab_ref[...] - base
        cols = lax.broadcasted_iota(jnp.int32, (T, V_TILE), 1)
        t_ref[...] += jnp.sum(
            jnp.where(cols == loc, logits, 0.0), axis=1, keepdims=True
        )

        @pl.when(step == N_TILES - 1)
        def _():
            comm_ref[:, 0:1] = s_ref[...]
            comm_ref[:, 1:2] = t_ref[...]
            rdma = pltpu.make_async_remote_copy(
                src_ref=comm_ref,
                dst_ref=recv_ref,
                send_sem=send_sem,
                recv_sem=recv_sem,
                device_id=(my_x, 1 - my_y),
                device_id_type=pl.DeviceIdType.MESH,
            )
            rdma.start()
            rdma.wait()
            s_tot = s_ref[...] + recv_ref[:, 0:1]
            t_tot = t_ref[...] + recv_ref[:, 1:2]
            out_ref[...] = jnp.log(s_tot) - t_tot

    out = pl.pallas_call(
        body,
        grid=(N_TILES,),
        in_specs=[
            pl.BlockSpec((T, D), lambda i: (0, 0)),
            pl.BlockSpec((D, V_TILE), lambda i: (0, i)),
            pl.BlockSpec((T, 1), lambda i: (0, 0)),
        ],
        out_specs=pl.BlockSpec((T, 1), lambda i: (0, 0)),
        out_shape=jax.ShapeDtypeStruct((T, 1), jnp.float32),
        scratch_shapes=[
            pltpu.VMEM((T, D), jnp.bfloat16),
            pltpu.VMEM((T, 1), jnp.float32),
            pltpu.VMEM((T, 1), jnp.float32),
            pltpu.VMEM((T, 2), jnp.float32),
            pltpu.VMEM((T, 2), jnp.float32),
            pltpu.SemaphoreType.DMA,
            pltpu.SemaphoreType.DMA,
        ],
        compiler_params=pltpu.CompilerParams(
            dimension_semantics=("arbitrary",),
        ),
    )(x, W, labels2d)
    return out.reshape(T)


# baseline (device time: 99185 ns/iter reference)
import jax
import jax.numpy as jnp
from jax import lax
from jax.experimental import pallas as pl
from jax.experimental.pallas import tpu as pltpu

T = 1024
D = 2048
V_LOCAL = 16384
V_TILE = 2048
N_TILES = V_LOCAL // V_TILE


def kernel(x, W, labels):
    labels2d = labels.reshape(T, 1)

    def body(x_ref, w_ref, lab_ref, out_ref,
             xb_ref, s_ref, t_ref, comm_ref, recv_ref,
             send_sem, recv_sem):
        step = pl.program_id(0)
        my_x = lax.axis_index("x")
        my_y = lax.axis_index("y")

        @pl.when(step == 0)
        def _():
            barrier_sem = pltpu.get_barrier_semaphore()
            pl.semaphore_signal(
                barrier_sem, inc=1,
                device_id=(my_x, 1 - my_y),
                device_id_type=pl.DeviceIdType.MESH,
            )
            pl.semaphore_wait(barrier_sem, 1)
            xb_ref[...] = x_ref[...].astype(jnp.bfloat16)
            s_ref[...] = jnp.zeros((T, 1), jnp.float32)
            t_ref[...] = jnp.zeros((T, 1), jnp.float32)

        logits = jnp.dot(
            xb_ref[...],
            w_ref[...].astype(jnp.bfloat16),
            preferred_element_type=jnp.float32,
        )

        s_ref[...] += jnp.sum(jnp.exp(logits), axis=1, keepdims=True)

        base = my_y * V_LOCAL + step * V_TILE
        loc = lab_ref[...] - base
        cols = lax.broadcasted_iota(jnp.int32, (T, V_TILE), 1)
        t_ref[...] += jnp.sum(
            jnp.where(cols == loc, logits, 0.0), axis=1, keepdims=True
        )

        @pl.when(step == N_TILES - 1)
        def _():
            comm_ref[:, 0:1] = s_ref[...]
            comm_ref[:, 1:2] = t_ref[...]
            rdma = pltpu.make_async_remote_copy(
                src_ref=comm_ref,
                dst_ref=recv_ref,
                send_sem=send_sem,
                recv_sem=recv_sem,
                device_id=(my_x, 1 - my_y),
                device_id_type=pl.DeviceIdType.MESH,
            )
            rdma.start()
            rdma.wait()
            s_tot = s_ref[...] + recv_ref[:, 0:1]
            t_tot = t_ref[...] + recv_ref[:, 1:2]
            out_ref[...] = jnp.log(s_tot) - t_tot

    out = pl.pallas_call(
        body,
        grid=(N_TILES,),
        in_specs=[
            pl.BlockSpec((T, D), lambda i: (0, 0)),
            pl.BlockSpec((D, V_TILE), lambda i: (0, i)),
            pl.BlockSpec((T, 1), lambda i: (0, 0)),
        ],
        out_specs=pl.BlockSpec((T, 1), lambda i: (0, 0)),
        out_shape=jax.ShapeDtypeStruct((T, 1), jnp.float32),
        scratch_shapes=[
            pltpu.VMEM((T, D), jnp.bfloat16),
            pltpu.VMEM((T, 1), jnp.float32),
            pltpu.VMEM((T, 1), jnp.float32),
            pltpu.VMEM((T, 2), jnp.float32),
            pltpu.VMEM((T, 2), jnp.float32),
            pltpu.SemaphoreType.DMA,
            pltpu.SemaphoreType.DMA,
        ],
        compiler_params=pltpu.CompilerParams(
            dimension_semantics=("arbitrary",),
            vmem_limit_bytes=96 * 1024 * 1024,
            collective_id=0,
        ),
    )(x, W, labels2d)
    return out.reshape(T)


# device time: 69790 ns/iter; 1.4212x vs baseline; 1.4212x over previous
import jax
import jax.numpy as jnp
from jax import lax
from jax.experimental import pallas as pl
from jax.experimental.pallas import tpu as pltpu

T = 1024
D = 2048
V_LOCAL = 16384
V_TILE = 1024
N_TILES = (V_LOCAL // 2) // V_TILE


def kernel(x, W, labels):
    labels2d = labels.reshape(T, 1)

    def body(x_ref, w_hbm, lab_ref, out_ref,
             xb_ref, wbuf, sx, rx, sy, ry, wsem, csem):
        my_x = lax.axis_index("x")
        my_y = lax.axis_index("y")

        barrier_sem = pltpu.get_barrier_semaphore()
        pl.semaphore_signal(
            barrier_sem, inc=1, device_id=(1 - my_x, my_y),
            device_id_type=pl.DeviceIdType.MESH,
        )
        pl.semaphore_signal(
            barrier_sem, inc=1, device_id=(my_x, 1 - my_y),
            device_id_type=pl.DeviceIdType.MESH,
        )
        pl.semaphore_wait(barrier_sem, 2)

        col0 = my_x * (N_TILES * V_TILE)

        def tile_copy(slot, t):
            return pltpu.make_async_copy(
                w_hbm.at[:, pl.ds(col0 + t * V_TILE, V_TILE)],
                wbuf.at[slot],
                wsem.at[slot],
            )

        tile_copy(0, 0).start()
        xb_ref[...] = x_ref[...].astype(jnp.bfloat16)

        s_acc = jnp.zeros((T, 1), jnp.float32)
        t_acc = jnp.zeros((T, 1), jnp.float32)
        cols = lax.broadcasted_iota(jnp.int32, (T, V_TILE), 1)
        for t in range(N_TILES):
            slot = t % 2
            if t + 1 < N_TILES:
                tile_copy(1 - slot, t + 1).start()
            tile_copy(slot, t).wait()
            logits = jnp.dot(
                xb_ref[...],
                wbuf[slot].astype(jnp.bfloat16),
                preferred_element_type=jnp.float32,
            )
            s_acc += jnp.sum(jnp.exp(logits), axis=1, keepdims=True)
            base = my_y * V_LOCAL + col0 + t * V_TILE
            loc = lab_ref[...] - base
            t_acc += jnp.sum(
                jnp.where(cols == loc, logits, 0.0), axis=1, keepdims=True
            )

        sx[:, 0:1] = s_acc
        sx[:, 1:2] = t_acc
        rdma_x = pltpu.make_async_remote_copy(
            src_ref=sx, dst_ref=rx,
            send_sem=csem.at[0], recv_sem=csem.at[1],
            device_id=(1 - my_x, my_y),
            device_id_type=pl.DeviceIdType.MESH,
        )
        rdma_x.start()
        rdma_x.wait()
        s_acc = s_acc + rx[:, 0:1]
        t_acc = t_acc + rx[:, 1:2]

        sy[:, 0:1] = s_acc
        sy[:, 1:2] = t_acc
        rdma_y = pltpu.make_async_remote_copy(
            src_ref=sy, dst_ref=ry,
            send_sem=csem.at[2], recv_sem=csem.at[3],
            device_id=(my_x, 1 - my_y),
            device_id_type=pl.DeviceIdType.MESH,
        )
        rdma_y.start()
        rdma_y.wait()
        s_tot = s_acc + ry[:, 0:1]
        t_tot = t_acc + ry[:, 1:2]
        out_ref[...] = jnp.log(s_tot) - t_tot

    out = pl.pallas_call(
        body,
        in_specs=[
            pl.BlockSpec(memory_space=pltpu.VMEM),
            pl.BlockSpec(memory_space=pltpu.MemorySpace.HBM),
            pl.BlockSpec(memory_space=pltpu.VMEM),
        ],
        out_specs=pl.BlockSpec(memory_space=pltpu.VMEM),
        out_shape=jax.ShapeDtypeStruct((T, 1), jnp.float32),
        scratch_shapes=[
            pltpu.VMEM((T, D), jnp.bfloat16),
            pltpu.VMEM((2, D, V_TILE), jnp.float32),
            pltpu.VMEM((T, 2), jnp.float32),
            pltpu.VMEM((T, 2), jnp.float32),
            pltpu.VMEM((T, 2), jnp.float32),
            pltpu.VMEM((T, 2), jnp.float32),
            pltpu.SemaphoreType.DMA((2,)),
            pltpu.SemaphoreType.DMA((4,)),
        ],
        compiler_params=pltpu.CompilerParams(
            vmem_limit_bytes=96 * 1024 * 1024,
            collective_id=0,
        ),
    )(x, W, labels2d)
    return out.reshape(T)
